# baseline (speedup 1.0000x reference)
import functools

import jax
import jax.numpy as jnp
from jax import lax
from jax.experimental import pallas as pl
from jax.experimental.pallas import tpu as pltpu
from jax.experimental.pallas import tpu_sc as plsc

DIM = 64
LANES = 16
NUM_CORES = 2
NUM_SUBCORES = 16
NW = NUM_CORES * NUM_SUBCORES
CHUNK = 512


def _sc_kernel(n_rows, ids_hbm, tg_hbm, emb_hbm, table_hbm, out_hbm,
               idx_v, tg_v, shift_v, emb_v, sem):
    c = lax.axis_index("c")
    s = lax.axis_index("s")
    wid = s * NUM_CORES + c
    rows_per_w = n_rows // NW
    nchunks = rows_per_w // CHUNK

    def chunk_body(ci, carry):
        base = wid * rows_per_w + ci * CHUNK
        pltpu.sync_copy(ids_hbm.at[pl.ds(base, CHUNK)], idx_v)
        gather = pltpu.async_copy(table_hbm.at[idx_v], shift_v, sem)
        pltpu.sync_copy(tg_hbm.at[pl.ds(base, CHUNK)], tg_v)
        pltpu.sync_copy(emb_hbm.at[pl.ds(base, CHUNK)], emb_v)
        gather.wait()

        def block_body(rb, carry2):
            r0 = rb * LANES
            tgv = tg_v[pl.ds(r0, LANES)]
            for j in range(LANES):
                i = r0 + j
                tgb = jnp.full((LANES,), tgv[j], jnp.float32)
                for k in range(DIM // LANES):
                    sh = shift_v[i, pl.ds(k * LANES, LANES)]
                    e = emb_v[i, pl.ds(k * LANES, LANES)]
                    sig = 1.0 / (1.0 + jnp.exp(-(tgb * sh)))
                    emb_v[i, pl.ds(k * LANES, LANES)] = e * (1.0 + sig)
            return carry2

        lax.fori_loop(0, CHUNK // LANES, block_body, 0)
        pltpu.sync_copy(emb_v, out_hbm.at[pl.ds(base, CHUNK)])
        return carry

    lax.fori_loop(0, nchunks, chunk_body, 0)


@jax.jit
def _dspp_sc(ids_flat, tg_flat, emb_flat, table):
    n_rows = ids_flat.shape[0]
    mesh = plsc.VectorSubcoreMesh(core_axis_name="c", subcore_axis_name="s")
    run = pl.kernel(
        functools.partial(_sc_kernel, n_rows),
        out_type=jax.ShapeDtypeStruct((n_rows, DIM), jnp.float32),
        mesh=mesh,
        scratch_types=[
            pltpu.VMEM((CHUNK,), jnp.int32),
            pltpu.VMEM((CHUNK,), jnp.float32),
            pltpu.VMEM((CHUNK, DIM), jnp.float32),
            pltpu.VMEM((CHUNK, DIM), jnp.float32),
            pltpu.SemaphoreType.DMA,
        ],
        compiler_params=pltpu.CompilerParams(use_tc_tiling_on_sc=False),
    )
    return run(ids_flat, tg_flat, emb_flat, table)


def kernel(id, emb, time_gap, user_shift_table):
    B, L = id.shape
    n = B * L
    out = _dspp_sc(
        id.reshape(n).astype(jnp.int32),
        time_gap.reshape(n),
        emb.reshape(n, DIM),
        user_shift_table,
    )
    return out.reshape(B, L, DIM)
